# baseline (speedup 1.0000x reference)
import functools

import jax
import jax.numpy as jnp
from jax.experimental import pallas as pl
from jax.experimental.pallas import tpu as pltpu
from jax.experimental.pallas import tpu_sc as plsc

D = 64
GATHER_W = 128
TC_BLOCK = 2048
N_CHUNKS = 8


def _sc_gather(table_pairs, idx_pair, Tc):
    mesh = plsc.VectorSubcoreMesh(core_axis_name="core", subcore_axis_name="subcore")

    @functools.partial(
        pl.kernel,
        out_type=jax.ShapeDtypeStruct((Tc, 2 * D), jnp.float32),
        mesh=mesh,
    )
    def gather_kernel(tbl_hbm, idx_hbm, out_hbm):
        def body(i_vmem, o_vmem):
            pltpu.sync_copy(tbl_hbm.at[i_vmem.at[0]], o_vmem)

        pltpu.emit_pipeline(
            body,
            grid=(Tc // GATHER_W,),
            in_specs=[pl.BlockSpec((1, GATHER_W), index_map=lambda i: (0, i))],
            out_specs=[pl.BlockSpec((GATHER_W, 2 * D), index_map=lambda i: (i, 0))],
            core_axis_name=("core", "subcore"),
            dimension_semantics=(pltpu.PARALLEL,),
        )(idx_hbm, out_hbm)

    return gather_kernel(table_pairs, idx_pair)


def _highway_body(emb_ref, par_ref, w0, b0, w1, b1, _prev_ref, out_ref):
    e = emb_ref[...]
    p = par_ref[...]
    h = jnp.where(p > 0, e[:, D:], e[:, :D])
    for (w, b) in ((w0, b0), (w1, b1)):
        pre = jnp.dot(h.astype(jnp.bfloat16), w[...],
                      preferred_element_type=jnp.float32) + b[...]
        t = jnp.maximum(pre[:, :D], 0.0)
        g = jax.nn.sigmoid(pre[:, D:])
        h = g * t + (1.0 - g) * h
    out_ref[...] = h


def _tc_highway_chunk(emb128, parity, weights, prev_out, chunk_block0, T):
    Tc = emb128.shape[0]
    wspec = pl.BlockSpec((D, 2 * D), lambda i: (0, 0))
    bspec = pl.BlockSpec((1, 2 * D), lambda i: (0, 0))
    kwargs = {}
    if prev_out is None:
        prev_out = jnp.zeros((1, 1), jnp.float32)
    else:
        kwargs["input_output_aliases"] = {6: 0}
    return pl.pallas_call(
        _highway_body,
        grid=(Tc // TC_BLOCK,),
        in_specs=[pl.BlockSpec((TC_BLOCK, 2 * D), lambda i: (i, 0)),
                  pl.BlockSpec((TC_BLOCK, 1), lambda i: (i, 0)),
                  wspec, bspec, wspec, bspec,
                  pl.BlockSpec(memory_space=pl.ANY)],
        out_specs=pl.BlockSpec((TC_BLOCK, D),
                               lambda i, c0=chunk_block0: (c0 + i, 0)),
        out_shape=jax.ShapeDtypeStruct((T, D), jnp.float32),
        **kwargs,
    )(emb128, parity, *weights, prev_out)


def kernel(batch, table, Wt0, bt0, Wg0, bg0, Wt1, bt1, Wg1, bg1):
    B, L = batch.shape
    T = B * L
    V = table.shape[0]
    Tc = T // N_CHUNKS

    idx = batch.reshape(-1).astype(jnp.int32)
    table_pairs = table.reshape(V // 2, 2 * D)
    idx_pair = (idx >> 1).reshape(N_CHUNKS, 1, Tc)
    parity = (idx & 1).reshape(N_CHUNKS, Tc, 1)

    w0 = jnp.concatenate([Wt0, Wg0], axis=1).astype(jnp.bfloat16)
    b0 = jnp.concatenate([bt0, bg0]).reshape(1, 2 * D)
    w1 = jnp.concatenate([Wt1, Wg1], axis=1).astype(jnp.bfloat16)
    b1 = jnp.concatenate([bt1, bg1]).reshape(1, 2 * D)
    weights = (w0, b0, w1, b1)

    out = jnp.zeros((0,))
    prev = None
    for c in range(N_CHUNKS):
        emb128 = _sc_gather(table_pairs, idx_pair[c], Tc)
        out = _tc_highway_chunk(emb128, parity[c], weights, prev,
                                c * (Tc // TC_BLOCK), T)
        prev = out
    return out.reshape(B, L, D)


# speedup vs baseline: 1.4082x; 1.4082x over previous
import functools

import jax
import jax.numpy as jnp
from jax.experimental import pallas as pl
from jax.experimental.pallas import tpu as pltpu
from jax.experimental.pallas import tpu_sc as plsc

D = 64
GATHER_W = 128
TC_ROWS = 1024
N_CHUNKS = 8


def _sc_gather(table, idx_chunk, Tc):
    mesh = plsc.VectorSubcoreMesh(core_axis_name="core", subcore_axis_name="subcore")

    @functools.partial(
        pl.kernel,
        out_type=jax.ShapeDtypeStruct((Tc, D), jnp.float32),
        mesh=mesh,
        compiler_params=pltpu.CompilerParams(use_tc_tiling_on_sc=False),
    )
    def gather_kernel(tbl_hbm, idx_hbm, out_hbm):
        def body(i_vmem, o_vmem):
            pltpu.sync_copy(tbl_hbm.at[i_vmem.at[0]], o_vmem)

        pltpu.emit_pipeline(
            body,
            grid=(Tc // GATHER_W,),
            in_specs=[pl.BlockSpec((1, GATHER_W), index_map=lambda i: (0, i))],
            out_specs=[pl.BlockSpec((GATHER_W, D), index_map=lambda i: (i, 0))],
            core_axis_name=("core", "subcore"),
            dimension_semantics=(pltpu.PARALLEL,),
        )(idx_hbm, out_hbm)

    return gather_kernel(table, idx_chunk).reshape(Tc // 2, 2 * D)


def _highway_body(emb_ref, w0, b0, w1, b1, _prev_ref, out_ref):
    h = emb_ref[...]
    for (w, b) in ((w0, b0), (w1, b1)):
        pre = jnp.dot(h.astype(jnp.bfloat16), w[...],
                      preferred_element_type=jnp.float32) + b[...]
        t = jnp.maximum(pre[:, :2 * D], 0.0)
        g = jax.nn.sigmoid(pre[:, 2 * D:])
        h = g * t + (1.0 - g) * h
    out_ref[...] = jnp.stack([h[:, :D], h[:, D:]], axis=1).reshape(2 * TC_ROWS, D)


def _tc_highway_chunk(emb128, weights, prev_out, chunk_block0, T):
    rows = emb128.shape[0]
    wspec = pl.BlockSpec((2 * D, 4 * D), lambda i: (0, 0))
    bspec = pl.BlockSpec((1, 4 * D), lambda i: (0, 0))
    kwargs = {}
    if prev_out is None:
        prev_out = jnp.zeros((1, 1), jnp.float32)
    else:
        kwargs["input_output_aliases"] = {5: 0}
    return pl.pallas_call(
        _highway_body,
        grid=(rows // TC_ROWS,),
        in_specs=[pl.BlockSpec((TC_ROWS, 2 * D), lambda i: (i, 0)),
                  wspec, bspec, wspec, bspec,
                  pl.BlockSpec(memory_space=pl.ANY)],
        out_specs=pl.BlockSpec((2 * TC_ROWS, D),
                               lambda i, c0=chunk_block0: (c0 + i, 0)),
        out_shape=jax.ShapeDtypeStruct((T, D), jnp.float32),
        **kwargs,
    )(emb128, *weights, prev_out)


def _block_diag_weights(Wt, bt, Wg, bg):
    z = jnp.zeros((D, D), jnp.float32)
    w = jnp.block([[Wt, z, Wg, z], [z, Wt, z, Wg]]).astype(jnp.bfloat16)
    b = jnp.concatenate([bt, bt, bg, bg]).reshape(1, 4 * D)
    return w, b


def kernel(batch, table, Wt0, bt0, Wg0, bg0, Wt1, bt1, Wg1, bg1):
    B, L = batch.shape
    T = B * L
    Tc = T // N_CHUNKS

    idx = batch.reshape(N_CHUNKS, 1, Tc).astype(jnp.int32)
    w0, b0 = _block_diag_weights(Wt0, bt0, Wg0, bg0)
    w1, b1 = _block_diag_weights(Wt1, bt1, Wg1, bg1)
    weights = (w0, b0, w1, b1)

    out = None
    for c in range(N_CHUNKS):
        emb128 = _sc_gather(table, idx[c], Tc)
        out = _tc_highway_chunk(emb128, weights, out,
                                c * (Tc // TC_ROWS // 2), T)
    return out.reshape(B, L, D)


# speedup vs baseline: 1.6422x; 1.1661x over previous
import functools

import jax
import jax.numpy as jnp
from jax.experimental import pallas as pl
from jax.experimental.pallas import tpu as pltpu
from jax.experimental.pallas import tpu_sc as plsc

D = 64
PREP_BV = 8192
GATHER_W = 128
TC_BLOCK = 2048
N_CHUNKS = 8


def _prep_body(tt_ref, out_ref):
    out_ref[:, :D] = jnp.transpose(tt_ref[...], (1, 0))


def _prep_table(tt, V):
    return pl.pallas_call(
        _prep_body,
        grid=(pl.cdiv(V, PREP_BV),),
        in_specs=[pl.BlockSpec((D, PREP_BV), lambda i: (0, i))],
        out_specs=pl.BlockSpec((PREP_BV, 2 * D), lambda i: (i, 0)),
        out_shape=jax.ShapeDtypeStruct((V, 2 * D), jnp.float32),
    )(tt)


def _sc_gather(table_pad, idx_chunk, Tc):
    mesh = plsc.VectorSubcoreMesh(core_axis_name="core", subcore_axis_name="subcore")

    @functools.partial(
        pl.kernel,
        out_type=jax.ShapeDtypeStruct((Tc, 2 * D), jnp.float32),
        mesh=mesh,
    )
    def gather_kernel(tbl_hbm, idx_hbm, out_hbm):
        def body(i_vmem, o_vmem):
            pltpu.sync_copy(tbl_hbm.at[i_vmem.at[0]], o_vmem)

        pltpu.emit_pipeline(
            body,
            grid=(Tc // GATHER_W,),
            in_specs=[pl.BlockSpec((1, GATHER_W), index_map=lambda i: (0, i))],
            out_specs=[pl.BlockSpec((GATHER_W, 2 * D), index_map=lambda i: (i, 0))],
            core_axis_name=("core", "subcore"),
            dimension_semantics=(pltpu.PARALLEL,),
        )(idx_hbm, out_hbm)

    return gather_kernel(table_pad, idx_chunk)


def _highway_body(emb_ref, w0, b0, w1, b1, _prev_ref, out_ref):
    e = emb_ref[...]
    h = e[:, :D]
    for (w, b) in ((w0, b0), (w1, b1)):
        pre = jnp.dot(h.astype(jnp.bfloat16), w[...],
                      preferred_element_type=jnp.float32) + b[...]
        t = jnp.maximum(pre[:, :D], 0.0)
        g = jax.nn.sigmoid(pre[:, D:])
        h = g * t + (1.0 - g) * h
    out_ref[...] = h


def _tc_highway_chunk(emb128, weights, prev_out, chunk_block0, T):
    Tc = emb128.shape[0]
    wspec = pl.BlockSpec((D, 2 * D), lambda i: (0, 0))
    bspec = pl.BlockSpec((1, 2 * D), lambda i: (0, 0))
    kwargs = {}
    if prev_out is None:
        prev_out = jnp.zeros((1, 1), jnp.float32)
    else:
        kwargs["input_output_aliases"] = {5: 0}
    return pl.pallas_call(
        _highway_body,
        grid=(Tc // TC_BLOCK,),
        in_specs=[pl.BlockSpec((TC_BLOCK, 2 * D), lambda i: (i, 0)),
                  wspec, bspec, wspec, bspec,
                  pl.BlockSpec(memory_space=pl.ANY)],
        out_specs=pl.BlockSpec((TC_BLOCK, D),
                               lambda i, c0=chunk_block0: (c0 + i, 0)),
        out_shape=jax.ShapeDtypeStruct((T, D), jnp.float32),
        **kwargs,
    )(emb128, *weights, prev_out)


def kernel(batch, table, Wt0, bt0, Wg0, bg0, Wt1, bt1, Wg1, bg1):
    B, L = batch.shape
    T = B * L
    V = table.shape[0]
    Tc = T // N_CHUNKS

    idx = batch.reshape(N_CHUNKS, 1, Tc).astype(jnp.int32)
    table_pad = _prep_table(table.T, V)

    w0 = jnp.concatenate([Wt0, Wg0], axis=1).astype(jnp.bfloat16)
    b0 = jnp.concatenate([bt0, bg0]).reshape(1, 2 * D)
    w1 = jnp.concatenate([Wt1, Wg1], axis=1).astype(jnp.bfloat16)
    b1 = jnp.concatenate([bt1, bg1]).reshape(1, 2 * D)
    weights = (w0, b0, w1, b1)

    out = None
    for c in range(N_CHUNKS):
        emb128 = _sc_gather(table_pad, idx[c], Tc)
        out = _tc_highway_chunk(emb128, weights, out,
                                c * (Tc // TC_BLOCK), T)
    return out.reshape(B, L, D)


# speedup vs baseline: 2.4186x; 1.4728x over previous
import functools

import jax
import jax.numpy as jnp
from jax.experimental import pallas as pl
from jax.experimental.pallas import tpu as pltpu
from jax.experimental.pallas import tpu_sc as plsc

D = 64
PREP_BV = 8192
GATHER_W = 128
N_CHUNKS = 8


def _prep_body(tt_ref, out_ref):
    out_ref[:, :D] = jnp.transpose(tt_ref[...], (1, 0))


def _prep_table(tt, V):
    return pl.pallas_call(
        _prep_body,
        grid=(pl.cdiv(V, PREP_BV),),
        in_specs=[pl.BlockSpec((D, PREP_BV), lambda i: (0, i))],
        out_specs=pl.BlockSpec((PREP_BV, 2 * D), lambda i: (i, 0)),
        out_shape=jax.ShapeDtypeStruct((V, 2 * D), jnp.float32),
    )(tt)


def _sc_gather(table_pad, idx_chunk, Tc):
    mesh = plsc.VectorSubcoreMesh(core_axis_name="core", subcore_axis_name="subcore")

    @functools.partial(
        pl.kernel,
        out_type=jax.ShapeDtypeStruct((Tc, 2 * D), jnp.float32),
        mesh=mesh,
    )
    def gather_kernel(tbl_hbm, idx_hbm, out_hbm):
        def body(i_vmem, o_vmem):
            pltpu.sync_copy(tbl_hbm.at[i_vmem.at[0]], o_vmem)

        pltpu.emit_pipeline(
            body,
            grid=(Tc // GATHER_W,),
            in_specs=[pl.BlockSpec((1, GATHER_W), index_map=lambda i: (0, i))],
            out_specs=[pl.BlockSpec((GATHER_W, 2 * D), index_map=lambda i: (i, 0))],
            core_axis_name=("core", "subcore"),
            dimension_semantics=(pltpu.PARALLEL,),
        )(idx_hbm, out_hbm)

    return gather_kernel(table_pad, idx_chunk)


def _highway_body(emb_ref, w0, b0, w1, b1, _prev_ref, out_ref, *, B):
    e = emb_ref[...]
    h = jnp.transpose(e[:, :D], (1, 0))
    for (w, b) in ((w0, b0), (w1, b1)):
        pre = jnp.dot(w[...], h.astype(jnp.bfloat16),
                      preferred_element_type=jnp.float32) + b[...]
        t = jnp.maximum(pre[:D, :], 0.0)
        g = jax.nn.sigmoid(pre[D:, :])
        h = g * t + (1.0 - g) * h
    out_ref[0, :, :] = h


def _tc_highway_chunk(emb128, weights, prev_out, l0, L, B):
    Tc = emb128.shape[0]
    steps = Tc // B
    wspec = pl.BlockSpec((2 * D, D), lambda i: (0, 0))
    bspec = pl.BlockSpec((2 * D, 1), lambda i: (0, 0))
    kwargs = {}
    if prev_out is None:
        prev_out = jnp.zeros((1, 1), jnp.float32)
    else:
        kwargs["input_output_aliases"] = {5: 0}
    return pl.pallas_call(
        functools.partial(_highway_body, B=B),
        grid=(steps,),
        in_specs=[pl.BlockSpec((B, 2 * D), lambda i: (i, 0)),
                  wspec, bspec, wspec, bspec,
                  pl.BlockSpec(memory_space=pl.ANY)],
        out_specs=pl.BlockSpec((1, D, B), lambda i, l0=l0: (l0 + i, 0, 0)),
        out_shape=jax.ShapeDtypeStruct((L, D, B), jnp.float32),
        **kwargs,
    )(emb128, *weights, prev_out)


def kernel(batch, table, Wt0, bt0, Wg0, bg0, Wt1, bt1, Wg1, bg1):
    B, L = batch.shape
    T = B * L
    V = table.shape[0]
    Tc = T // N_CHUNKS
    Lc = L // N_CHUNKS

    idx = batch.T.reshape(N_CHUNKS, 1, Tc).astype(jnp.int32)
    table_pad = _prep_table(table.T, V)

    def stacked(Wt, bt, Wg, bg):
        w = jnp.concatenate([Wt.T, Wg.T], axis=0).astype(jnp.bfloat16)
        b = jnp.concatenate([bt, bg]).reshape(2 * D, 1)
        return w, b

    w0, b0 = stacked(Wt0, bt0, Wg0, bg0)
    w1, b1 = stacked(Wt1, bt1, Wg1, bg1)
    weights = (w0, b0, w1, b1)

    out = None
    for c in range(N_CHUNKS):
        emb128 = _sc_gather(table_pad, idx[c], Tc)
        out = _tc_highway_chunk(emb128, weights, out, c * Lc, L, B)
    return jnp.transpose(out, (2, 0, 1))


# speedup vs baseline: 2.4394x; 1.0086x over previous
import functools

import jax
import jax.numpy as jnp
from jax.experimental import pallas as pl
from jax.experimental.pallas import tpu as pltpu
from jax.experimental.pallas import tpu_sc as plsc

D = 64
PREP_BV = 8192
GATHER_W = 128
N_CHUNKS = 8


def _prep_body(tt_ref, out_ref):
    out_ref[:, :D] = jnp.transpose(tt_ref[...], (1, 0))


def _prep_table(tt, V):
    return pl.pallas_call(
        _prep_body,
        grid=(pl.cdiv(V, PREP_BV),),
        in_specs=[pl.BlockSpec((D, PREP_BV), lambda i: (0, i))],
        out_specs=pl.BlockSpec((PREP_BV, 2 * D), lambda i: (i, 0)),
        out_shape=jax.ShapeDtypeStruct((V, 2 * D), jnp.float32),
    )(tt)


def _sc_gather(table_pad, idx_chunk, Tc):
    mesh = plsc.VectorSubcoreMesh(core_axis_name="core", subcore_axis_name="subcore")

    @functools.partial(
        pl.kernel,
        out_type=jax.ShapeDtypeStruct((Tc, 2 * D), jnp.float32),
        mesh=mesh,
    )
    def gather_kernel(tbl_hbm, idx_hbm, out_hbm):
        def body(i_vmem, o_vmem):
            pltpu.sync_copy(tbl_hbm.at[i_vmem.at[0]], o_vmem)

        pltpu.emit_pipeline(
            body,
            grid=(Tc // GATHER_W,),
            in_specs=[pl.BlockSpec((1, GATHER_W), index_map=lambda i: (0, i))],
            out_specs=[pl.BlockSpec((GATHER_W, 2 * D), index_map=lambda i: (i, 0))],
            core_axis_name=("core", "subcore"),
            dimension_semantics=(pltpu.PARALLEL,),
        )(idx_hbm, out_hbm)

    return gather_kernel(table_pad, idx_chunk)


def _highway_body(emb_ref, w0, b0, w1, b1, _prev_ref, out_ref, *, B):
    e = emb_ref[...]
    h = jnp.transpose(e[:, :D].astype(jnp.bfloat16), (1, 0))
    one = jnp.bfloat16(1.0)
    for (w, b) in ((w0, b0), (w1, b1)):
        pre = (jnp.dot(w[...], h, preferred_element_type=jnp.float32)
               + b[...]).astype(jnp.bfloat16)
        t = jnp.maximum(pre[:D, :], jnp.bfloat16(0.0))
        g = jax.nn.sigmoid(pre[D:, :])
        h = g * t + (one - g) * h
    out_ref[0, :, :] = h.astype(jnp.float32)


def _tc_highway_chunk(emb128, weights, prev_out, l0, L, B):
    Tc = emb128.shape[0]
    steps = Tc // B
    wspec = pl.BlockSpec((2 * D, D), lambda i: (0, 0))
    bspec = pl.BlockSpec((2 * D, 1), lambda i: (0, 0))
    kwargs = {}
    if prev_out is None:
        prev_out = jnp.zeros((1, 1), jnp.float32)
    else:
        kwargs["input_output_aliases"] = {5: 0}
    return pl.pallas_call(
        functools.partial(_highway_body, B=B),
        grid=(steps,),
        in_specs=[pl.BlockSpec((B, 2 * D), lambda i: (i, 0)),
                  wspec, bspec, wspec, bspec,
                  pl.BlockSpec(memory_space=pl.ANY)],
        out_specs=pl.BlockSpec((1, D, B), lambda i, l0=l0: (l0 + i, 0, 0)),
        out_shape=jax.ShapeDtypeStruct((L, D, B), jnp.float32),
        **kwargs,
    )(emb128, *weights, prev_out)


def kernel(batch, table, Wt0, bt0, Wg0, bg0, Wt1, bt1, Wg1, bg1):
    B, L = batch.shape
    T = B * L
    V = table.shape[0]
    Tc = T // N_CHUNKS
    Lc = L // N_CHUNKS

    idx = batch.T.reshape(N_CHUNKS, 1, Tc).astype(jnp.int32)
    table_pad = _prep_table(table.T, V)

    def stacked(Wt, bt, Wg, bg):
        w = jnp.concatenate([Wt.T, Wg.T], axis=0).astype(jnp.bfloat16)
        b = jnp.concatenate([bt, bg]).reshape(2 * D, 1)
        return w, b

    w0, b0 = stacked(Wt0, bt0, Wg0, bg0)
    w1, b1 = stacked(Wt1, bt1, Wg1, bg1)
    weights = (w0, b0, w1, b1)

    out = None
    for c in range(N_CHUNKS):
        emb128 = _sc_gather(table_pad, idx[c], Tc)
        out = _tc_highway_chunk(emb128, weights, out, c * Lc, L, B)
    return jnp.transpose(out, (2, 0, 1))
